# Initial kernel scaffold; baseline (speedup 1.0000x reference)
import jax
import jax.numpy as jnp
from jax.experimental import pallas as pl


def kernel(x, edge_index, params):
    raise NotImplementedError("write your pallas kernel here")


# speedup vs baseline: 5.8995x; 5.8995x over previous
import functools

import jax
import jax.numpy as jnp
from jax import lax
from jax.experimental import pallas as pl
from jax.experimental.pallas import tpu as pltpu
from jax.experimental.pallas import tpu_sc as plsc

NN = 10000
NPAD = 10240
EE = 640000
RB = 1000
NR = NN // RB
K = 80
LAYER_DIMS = [3, 32, 64, 128, 256, 256, 512, 512, 256, 256, 128]


def _leaky(v):
    return jnp.where(v >= 0, v, 0.01 * v)


def _sc_info():
    info = plsc.get_sparse_core_info()
    return info.num_cores, info.num_subcores


def _seg_sum_sc(tables, src, dst, zeros, nc, dc):
    NC, NS = _sc_info()
    NW = NC * NS
    epw = EE // NW
    nchunks = epw // K
    rps = NPAD // NS
    mesh = plsc.VectorSubcoreMesh(core_axis_name="c", subcore_axis_name="s")

    @functools.partial(
        pl.kernel, mesh=mesh,
        out_type=[jax.ShapeDtypeStruct((NC * NPAD, dc), jnp.float32)] * nc,
        scratch_types=[
            pltpu.VMEM((K,), jnp.int32),
            pltpu.VMEM((K,), jnp.int32),
            pltpu.VMEM((K, dc), jnp.float32),
            pltpu.VMEM_SHARED((NPAD, dc), jnp.float32),
            pltpu.SemaphoreType.DMA,
        ])
    def k(*refs):
        table_refs = refs[:nc]
        src_ref, dst_ref, zero_ref = refs[nc], refs[nc + 1], refs[nc + 2]
        out_refs = refs[nc + 3:nc + 3 + nc]
        sidx, didx, rows, accum, sem = refs[nc + 3 + nc:]
        cid = lax.axis_index("c")
        sid = lax.axis_index("s")
        wid = sid * NC + cid
        base0 = wid * epw
        for c in range(nc):
            pltpu.sync_copy(zero_ref, accum.at[pl.ds(sid * rps, rps)])
            plsc.subcore_barrier()

            def body(j, carry):
                base = pl.multiple_of(base0 + j * K, 8)
                pltpu.sync_copy(src_ref.at[pl.ds(base, K)], sidx)
                pltpu.sync_copy(dst_ref.at[pl.ds(base, K)], didx)
                pltpu.async_copy(table_refs[c].at[sidx], rows, sem).wait()
                pltpu.sync_copy(rows, accum.at[didx], add=True)
                return carry

            lax.fori_loop(0, nchunks, body, 0)
            plsc.subcore_barrier()
            pltpu.sync_copy(
                accum.at[pl.ds(sid * rps, rps)],
                out_refs[c].at[pl.ds(cid * NPAD + sid * rps, rps)])

    return k(*tables, src, dst, zeros)


def _deg_sc(dst, ones, zeros):
    NC, NS = _sc_info()
    NW = NC * NS
    epw = EE // NW
    nchunks = epw // K
    rps = NPAD // NS
    mesh = plsc.VectorSubcoreMesh(core_axis_name="c", subcore_axis_name="s")

    @functools.partial(
        pl.kernel, mesh=mesh,
        out_type=jax.ShapeDtypeStruct((NC * NPAD, 128), jnp.float32),
        scratch_types=[
            pltpu.VMEM((K,), jnp.int32),
            pltpu.VMEM((K, 128), jnp.float32),
            pltpu.VMEM_SHARED((NPAD, 128), jnp.float32),
        ])
    def k(dst_ref, ones_ref, zero_ref, out_ref, didx, ones_v, accum):
        cid = lax.axis_index("c")
        sid = lax.axis_index("s")
        wid = sid * NC + cid
        base0 = wid * epw
        pltpu.sync_copy(ones_ref, ones_v)
        pltpu.sync_copy(zero_ref, accum.at[pl.ds(sid * rps, rps)])
        plsc.subcore_barrier()

        def body(j, carry):
            base = pl.multiple_of(base0 + j * K, 8)
            pltpu.sync_copy(dst_ref.at[pl.ds(base, K)], didx)
            pltpu.sync_copy(ones_v, accum.at[didx], add=True)
            return carry

        lax.fori_loop(0, nchunks, body, 0)
        plsc.subcore_barrier()
        pltpu.sync_copy(
            accum.at[pl.ds(sid * rps, rps)],
            out_ref.at[pl.ds(cid * NPAD + sid * rps, rps)])

    return k(dst, ones, zeros)


def _tc_dinv(degs):
    def body(d_ref, out_ref):
        s = d_ref[...]
        out_ref[...] = lax.rsqrt(s[0, :, 0:1] + s[1, :, 0:1] + 1.0)

    return pl.pallas_call(
        body,
        grid=(NR,),
        in_specs=[pl.BlockSpec((2, RB, 128), lambda r: (0, r, 0))],
        out_specs=pl.BlockSpec((RB, 1), lambda r: (r, 0)),
        out_shape=jax.ShapeDtypeStruct((NN, 1), jnp.float32),
    )(degs)


def _tc_first(x, W, dinv, dc):
    d_in = x.shape[1]
    d_out = W.shape[1]

    def body(x_ref, w_ref, dinv_ref, out_ref):
        g = jnp.dot(x_ref[...], w_ref[...], preferred_element_type=jnp.float32)
        g = g * dinv_ref[...]
        if d_out < dc:
            g = jnp.concatenate(
                [g, jnp.zeros((RB, dc - d_out), jnp.float32)], axis=1)
        out_ref[0] = g
    return pl.pallas_call(
        body,
        grid=(NR,),
        in_specs=[
            pl.BlockSpec((RB, d_in), lambda r: (r, 0)),
            pl.BlockSpec((d_in, d_out), lambda r: (0, 0)),
            pl.BlockSpec((RB, 1), lambda r: (r, 0)),
        ],
        out_specs=pl.BlockSpec((1, RB, dc), lambda r: (0, r, 0)),
        out_shape=jax.ShapeDtypeStruct((1, NN, dc), jnp.float32),
    )(x, W, dinv)


def _tc_mid(z, colsum, m2, gamma, beta, W, dinv, nc, dc):
    d_in = z.shape[1]
    d_out = W.shape[1]

    def body(z_ref, su_ref, m2_ref, g_ref, b_ref, w_ref, dinv_ref, out_ref):
        mu = su_ref[...] / NN
        var = m2_ref[...] / NN
        h = _leaky(g_ref[...] * (z_ref[...] - mu) * lax.rsqrt(var + 1e-5)
                   + b_ref[...])
        g = jnp.dot(h, w_ref[...], preferred_element_type=jnp.float32)
        g = g * dinv_ref[...]
        if d_out < nc * dc:
            g = jnp.concatenate(
                [g, jnp.zeros((RB, nc * dc - d_out), jnp.float32)], axis=1)
        for c in range(nc):
            out_ref[c] = g[:, c * dc:(c + 1) * dc]

    return pl.pallas_call(
        body,
        grid=(NR,),
        in_specs=[
            pl.BlockSpec((RB, d_in), lambda r: (r, 0)),
            pl.BlockSpec((1, d_in), lambda r: (0, 0)),
            pl.BlockSpec((1, d_in), lambda r: (0, 0)),
            pl.BlockSpec((1, d_in), lambda r: (0, 0)),
            pl.BlockSpec((1, d_in), lambda r: (0, 0)),
            pl.BlockSpec((d_in, d_out), lambda r: (0, 0)),
            pl.BlockSpec((RB, 1), lambda r: (r, 0)),
        ],
        out_specs=pl.BlockSpec((nc, RB, dc), lambda r: (0, r, 0)),
        out_shape=jax.ShapeDtypeStruct((nc, NN, dc), jnp.float32),
    )(z, colsum, m2, gamma, beta, W, dinv)


def _tc_combine(S_list, htil, dinv, bias, nc, dc, d):
    def body(*refs):
        S_refs = refs[:nc]
        ht_ref, dinv_ref, b_ref, z_ref, st_ref = refs[nc:]
        parts = []
        for c in range(nc):
            s = S_refs[c][...]
            parts.append(s[0] + s[1] + ht_ref[c])
        t = jnp.concatenate(parts, axis=1) if nc > 1 else parts[0]
        if d < nc * dc:
            t = t[:, :d]
        z = t * dinv_ref[...] + b_ref[...]
        z_ref[...] = z
        new = jnp.sum(z, axis=0, keepdims=True)

        @pl.when(pl.program_id(0) == 0)
        def _():
            st_ref[...] = new

        @pl.when(pl.program_id(0) != 0)
        def _():
            st_ref[...] = st_ref[...] + new

    return pl.pallas_call(
        body,
        grid=(NR,),
        in_specs=(
            [pl.BlockSpec((2, RB, dc), lambda r: (0, r, 0)) for _ in range(nc)]
            + [
                pl.BlockSpec((nc, RB, dc), lambda r: (0, r, 0)),
                pl.BlockSpec((RB, 1), lambda r: (r, 0)),
                pl.BlockSpec((1, d), lambda r: (0, 0)),
            ]),
        out_specs=[
            pl.BlockSpec((RB, d), lambda r: (r, 0)),
            pl.BlockSpec((1, d), lambda r: (0, 0)),
        ],
        out_shape=[
            jax.ShapeDtypeStruct((NN, d), jnp.float32),
            jax.ShapeDtypeStruct((1, d), jnp.float32),
        ],
    )(*S_list, htil, dinv, bias)


def _tc_m2(z, colsum):
    d = z.shape[1]

    def body(z_ref, su_ref, m2_ref):
        dv = z_ref[...] - su_ref[...] / NN
        new = jnp.sum(dv * dv, axis=0, keepdims=True)

        @pl.when(pl.program_id(0) == 0)
        def _():
            m2_ref[...] = new

        @pl.when(pl.program_id(0) != 0)
        def _():
            m2_ref[...] = m2_ref[...] + new

    return pl.pallas_call(
        body,
        grid=(NR,),
        in_specs=[
            pl.BlockSpec((RB, d), lambda r: (r, 0)),
            pl.BlockSpec((1, d), lambda r: (0, 0)),
        ],
        out_specs=pl.BlockSpec((1, d), lambda r: (0, 0)),
        out_shape=jax.ShapeDtypeStruct((1, d), jnp.float32),
    )(z, colsum)


def _tc_head(z, colsum, m2, gamma, beta, w1, b1, w2, b2):
    d_in = z.shape[1]
    dh = w1.shape[1]
    do = w2.shape[1]

    def body(z_ref, su_ref, m2_ref, g_ref, be_ref, w1_ref, b1_ref, w2_ref,
             b2_ref, out_ref):
        mu = su_ref[...] / NN
        var = m2_ref[...] / NN
        h = _leaky(g_ref[...] * (z_ref[...] - mu) * lax.rsqrt(var + 1e-5)
                   + be_ref[...])
        t = _leaky(
            jnp.dot(h, w1_ref[...], preferred_element_type=jnp.float32)
            + b1_ref[...])
        out_ref[...] = (
            jnp.dot(t, w2_ref[...], preferred_element_type=jnp.float32)
            + b2_ref[...])

    return pl.pallas_call(
        body,
        grid=(NR,),
        in_specs=[
            pl.BlockSpec((RB, d_in), lambda r: (r, 0)),
            pl.BlockSpec((1, d_in), lambda r: (0, 0)),
            pl.BlockSpec((1, d_in), lambda r: (0, 0)),
            pl.BlockSpec((1, d_in), lambda r: (0, 0)),
            pl.BlockSpec((1, d_in), lambda r: (0, 0)),
            pl.BlockSpec((d_in, dh), lambda r: (0, 0)),
            pl.BlockSpec((1, dh), lambda r: (0, 0)),
            pl.BlockSpec((dh, do), lambda r: (0, 0)),
            pl.BlockSpec((1, do), lambda r: (0, 0)),
        ],
        out_specs=pl.BlockSpec((RB, do), lambda r: (r, 0)),
        out_shape=jax.ShapeDtypeStruct((NN, do), jnp.float32),
    )(z, colsum, m2, gamma, beta, w1, b1, w2, b2)


def kernel(x, edge_index, params):
    f32 = jnp.float32
    src = edge_index[0]
    dst = edge_index[1]
    NC, NS = _sc_info()
    rps = NPAD // NS

    ones128 = jnp.ones((K, 128), f32)
    zeros128 = jnp.zeros((rps, 128), f32)
    degs = _deg_sc(dst, ones128, zeros128)
    dinv = _tc_dinv(degs.reshape(NC, NPAD, 128))

    z = None
    colsum = None
    m2 = None
    htil = None
    for i in range(10):
        d = LAYER_DIMS[i + 1]
        dc = 128
        nc = (d + dc - 1) // dc
        if i == 0:
            htil = _tc_first(x, params['W0'], dinv, dc)
        else:
            htil = _tc_mid(
                z, colsum, m2,
                params['g%d' % (i - 1)].reshape(1, -1),
                params['be%d' % (i - 1)].reshape(1, -1),
                params['W%d' % i], dinv, nc, dc)
        tables = [htil[c] for c in range(nc)]
        S = _seg_sum_sc(tables, src, dst, zeros128, nc, dc)
        if nc == 1:
            S = [S] if not isinstance(S, (list, tuple)) else list(S)
        S = [s.reshape(NC, NPAD, dc) for s in S]
        z, colsum = _tc_combine(
            S, htil, dinv, params['bW%d' % i].reshape(1, -1), nc, dc, d)
        m2 = _tc_m2(z, colsum)

    return _tc_head(
        z, colsum, m2,
        params['g9'].reshape(1, -1), params['be9'].reshape(1, -1),
        params['lin1_W'], params['lin1_b'].reshape(1, -1),
        params['lin2_W'], params['lin2_b'].reshape(1, -1))


# speedup vs baseline: 7.4452x; 1.2620x over previous
import functools

import jax
import jax.numpy as jnp
from jax import lax
from jax.experimental import pallas as pl
from jax.experimental.pallas import tpu as pltpu
from jax.experimental.pallas import tpu_sc as plsc

NN = 10000
NPAD = 10240
EE = 640000
RB = 1000
NR = NN // RB
K = 80
LAYER_DIMS = [3, 32, 64, 128, 256, 256, 512, 512, 256, 256, 128]


def _leaky(v):
    return jnp.where(v >= 0, v, 0.01 * v)


def _sc_info():
    info = plsc.get_sparse_core_info()
    return info.num_cores, info.num_subcores


def _seg_sum_sc(tables, src, dst, zeros, nc, dc):
    NC, NS = _sc_info()
    NW = NC * NS
    epw = EE // NW
    nchunks = epw // K
    rps = NPAD // NS
    mesh = plsc.VectorSubcoreMesh(core_axis_name="c", subcore_axis_name="s")

    npairs = nchunks // 2

    @functools.partial(
        pl.kernel, mesh=mesh,
        out_type=[jax.ShapeDtypeStruct((NC * NPAD, dc), jnp.float32)] * nc,
        scratch_types=[
            pltpu.VMEM((2 * K,), jnp.int32),
            pltpu.VMEM((2, K), jnp.int32),
            pltpu.VMEM((K, dc), jnp.float32),
            pltpu.VMEM((K, dc), jnp.float32),
            pltpu.VMEM_SHARED((NPAD, dc), jnp.float32),
            pltpu.SemaphoreType.DMA,
            pltpu.SemaphoreType.DMA,
        ])
    def k(*refs):
        table_refs = refs[:nc]
        src_ref, dst_ref, zero_ref = refs[nc], refs[nc + 1], refs[nc + 2]
        out_refs = refs[nc + 3:nc + 3 + nc]
        sidx, didx, rows0, rows1, accum, sem0, sem1 = refs[nc + 3 + nc:]
        cid = lax.axis_index("c")
        sid = lax.axis_index("s")
        wid = sid * NC + cid
        base0 = wid * epw
        for c in range(nc):
            pltpu.sync_copy(zero_ref, accum.at[pl.ds(sid * rps, rps)])
            plsc.subcore_barrier()
            table = table_refs[c]

            def body(j, carry):
                base = pl.multiple_of(base0 + j * (2 * K), 8)
                pltpu.sync_copy(src_ref.at[pl.ds(base, 2 * K)], sidx)
                pltpu.sync_copy(dst_ref.at[pl.ds(base, K)], didx.at[0])
                pltpu.sync_copy(dst_ref.at[pl.ds(base + K, K)], didx.at[1])
                pltpu.async_copy(table.at[sidx.at[pl.ds(0, K)]], rows0, sem0)
                pltpu.async_copy(table.at[sidx.at[pl.ds(K, K)]], rows1, sem1)
                pltpu.make_async_copy(
                    table.at[sidx.at[pl.ds(0, K)]], rows0, sem0).wait()
                pltpu.sync_copy(rows0, accum.at[didx.at[0]], add=True)
                pltpu.make_async_copy(
                    table.at[sidx.at[pl.ds(K, K)]], rows1, sem1).wait()
                pltpu.sync_copy(rows1, accum.at[didx.at[1]], add=True)
                return carry

            lax.fori_loop(0, npairs, body, 0)
            plsc.subcore_barrier()
            pltpu.sync_copy(
                accum.at[pl.ds(sid * rps, rps)],
                out_refs[c].at[pl.ds(cid * NPAD + sid * rps, rps)])

    return k(*tables, src, dst, zeros)


def _deg_sc(dst, ones, zeros):
    NC, NS = _sc_info()
    NW = NC * NS
    epw = EE // NW
    nchunks = epw // K
    rps = NPAD // NS
    mesh = plsc.VectorSubcoreMesh(core_axis_name="c", subcore_axis_name="s")

    @functools.partial(
        pl.kernel, mesh=mesh,
        out_type=jax.ShapeDtypeStruct((NC * NPAD, 128), jnp.float32),
        scratch_types=[
            pltpu.VMEM((K,), jnp.int32),
            pltpu.VMEM((K, 128), jnp.float32),
            pltpu.VMEM_SHARED((NPAD, 128), jnp.float32),
        ])
    def k(dst_ref, ones_ref, zero_ref, out_ref, didx, ones_v, accum):
        cid = lax.axis_index("c")
        sid = lax.axis_index("s")
        wid = sid * NC + cid
        base0 = wid * epw
        pltpu.sync_copy(ones_ref, ones_v)
        pltpu.sync_copy(zero_ref, accum.at[pl.ds(sid * rps, rps)])
        plsc.subcore_barrier()

        def body(j, carry):
            base = pl.multiple_of(base0 + j * K, 8)
            pltpu.sync_copy(dst_ref.at[pl.ds(base, K)], didx)
            pltpu.sync_copy(ones_v, accum.at[didx], add=True)
            return carry

        lax.fori_loop(0, nchunks, body, 0)
        plsc.subcore_barrier()
        pltpu.sync_copy(
            accum.at[pl.ds(sid * rps, rps)],
            out_ref.at[pl.ds(cid * NPAD + sid * rps, rps)])

    return k(dst, ones, zeros)


def _tc_dinv(degs):
    def body(d_ref, out_ref):
        s = d_ref[...]
        out_ref[...] = lax.rsqrt(s[0, :, 0:1] + s[1, :, 0:1] + 1.0)

    return pl.pallas_call(
        body,
        grid=(NR,),
        in_specs=[pl.BlockSpec((2, RB, 128), lambda r: (0, r, 0))],
        out_specs=pl.BlockSpec((RB, 1), lambda r: (r, 0)),
        out_shape=jax.ShapeDtypeStruct((NN, 1), jnp.float32),
    )(degs)


def _tc_first(x, W, dinv, dc):
    d_in = x.shape[1]
    d_out = W.shape[1]

    def body(x_ref, w_ref, dinv_ref, out_ref):
        g = jnp.dot(x_ref[...], w_ref[...], preferred_element_type=jnp.float32)
        g = g * dinv_ref[...]
        if d_out < dc:
            g = jnp.concatenate(
                [g, jnp.zeros((RB, dc - d_out), jnp.float32)], axis=1)
        out_ref[0] = g
    return pl.pallas_call(
        body,
        grid=(NR,),
        in_specs=[
            pl.BlockSpec((RB, d_in), lambda r: (r, 0)),
            pl.BlockSpec((d_in, d_out), lambda r: (0, 0)),
            pl.BlockSpec((RB, 1), lambda r: (r, 0)),
        ],
        out_specs=pl.BlockSpec((1, RB, dc), lambda r: (0, r, 0)),
        out_shape=jax.ShapeDtypeStruct((1, NN, dc), jnp.float32),
    )(x, W, dinv)


def _tc_mid(z, colsum, m2, gamma, beta, W, dinv, nc, dc):
    d_in = z.shape[1]
    d_out = W.shape[1]

    def body(z_ref, su_ref, m2_ref, g_ref, b_ref, w_ref, dinv_ref, out_ref):
        mu = su_ref[...] / NN
        var = m2_ref[...] / NN
        h = _leaky(g_ref[...] * (z_ref[...] - mu) * lax.rsqrt(var + 1e-5)
                   + b_ref[...])
        g = jnp.dot(h, w_ref[...], preferred_element_type=jnp.float32)
        g = g * dinv_ref[...]
        if d_out < nc * dc:
            g = jnp.concatenate(
                [g, jnp.zeros((RB, nc * dc - d_out), jnp.float32)], axis=1)
        for c in range(nc):
            out_ref[c] = g[:, c * dc:(c + 1) * dc]

    return pl.pallas_call(
        body,
        grid=(NR,),
        in_specs=[
            pl.BlockSpec((RB, d_in), lambda r: (r, 0)),
            pl.BlockSpec((1, d_in), lambda r: (0, 0)),
            pl.BlockSpec((1, d_in), lambda r: (0, 0)),
            pl.BlockSpec((1, d_in), lambda r: (0, 0)),
            pl.BlockSpec((1, d_in), lambda r: (0, 0)),
            pl.BlockSpec((d_in, d_out), lambda r: (0, 0)),
            pl.BlockSpec((RB, 1), lambda r: (r, 0)),
        ],
        out_specs=pl.BlockSpec((nc, RB, dc), lambda r: (0, r, 0)),
        out_shape=jax.ShapeDtypeStruct((nc, NN, dc), jnp.float32),
    )(z, colsum, m2, gamma, beta, W, dinv)


def _tc_combine(S_list, htil, dinv, bias, nc, dc, d):
    def body(*refs):
        S_refs = refs[:nc]
        ht_ref, dinv_ref, b_ref, z_ref, st_ref = refs[nc:]
        parts = []
        for c in range(nc):
            s = S_refs[c][...]
            parts.append(s[0] + s[1] + ht_ref[c])
        t = jnp.concatenate(parts, axis=1) if nc > 1 else parts[0]
        if d < nc * dc:
            t = t[:, :d]
        z = t * dinv_ref[...] + b_ref[...]
        z_ref[...] = z
        new = jnp.sum(z, axis=0, keepdims=True)

        @pl.when(pl.program_id(0) == 0)
        def _():
            st_ref[...] = new

        @pl.when(pl.program_id(0) != 0)
        def _():
            st_ref[...] = st_ref[...] + new

    return pl.pallas_call(
        body,
        grid=(NR,),
        in_specs=(
            [pl.BlockSpec((2, RB, dc), lambda r: (0, r, 0)) for _ in range(nc)]
            + [
                pl.BlockSpec((nc, RB, dc), lambda r: (0, r, 0)),
                pl.BlockSpec((RB, 1), lambda r: (r, 0)),
                pl.BlockSpec((1, d), lambda r: (0, 0)),
            ]),
        out_specs=[
            pl.BlockSpec((RB, d), lambda r: (r, 0)),
            pl.BlockSpec((1, d), lambda r: (0, 0)),
        ],
        out_shape=[
            jax.ShapeDtypeStruct((NN, d), jnp.float32),
            jax.ShapeDtypeStruct((1, d), jnp.float32),
        ],
    )(*S_list, htil, dinv, bias)


def _tc_m2(z, colsum):
    d = z.shape[1]

    def body(z_ref, su_ref, m2_ref):
        dv = z_ref[...] - su_ref[...] / NN
        new = jnp.sum(dv * dv, axis=0, keepdims=True)

        @pl.when(pl.program_id(0) == 0)
        def _():
            m2_ref[...] = new

        @pl.when(pl.program_id(0) != 0)
        def _():
            m2_ref[...] = m2_ref[...] + new

    return pl.pallas_call(
        body,
        grid=(NR,),
        in_specs=[
            pl.BlockSpec((RB, d), lambda r: (r, 0)),
            pl.BlockSpec((1, d), lambda r: (0, 0)),
        ],
        out_specs=pl.BlockSpec((1, d), lambda r: (0, 0)),
        out_shape=jax.ShapeDtypeStruct((1, d), jnp.float32),
    )(z, colsum)


def _tc_head(z, colsum, m2, gamma, beta, w1, b1, w2, b2):
    d_in = z.shape[1]
    dh = w1.shape[1]
    do = w2.shape[1]

    def body(z_ref, su_ref, m2_ref, g_ref, be_ref, w1_ref, b1_ref, w2_ref,
             b2_ref, out_ref):
        mu = su_ref[...] / NN
        var = m2_ref[...] / NN
        h = _leaky(g_ref[...] * (z_ref[...] - mu) * lax.rsqrt(var + 1e-5)
                   + be_ref[...])
        t = _leaky(
            jnp.dot(h, w1_ref[...], preferred_element_type=jnp.float32)
            + b1_ref[...])
        out_ref[...] = (
            jnp.dot(t, w2_ref[...], preferred_element_type=jnp.float32)
            + b2_ref[...])

    return pl.pallas_call(
        body,
        grid=(NR,),
        in_specs=[
            pl.BlockSpec((RB, d_in), lambda r: (r, 0)),
            pl.BlockSpec((1, d_in), lambda r: (0, 0)),
            pl.BlockSpec((1, d_in), lambda r: (0, 0)),
            pl.BlockSpec((1, d_in), lambda r: (0, 0)),
            pl.BlockSpec((1, d_in), lambda r: (0, 0)),
            pl.BlockSpec((d_in, dh), lambda r: (0, 0)),
            pl.BlockSpec((1, dh), lambda r: (0, 0)),
            pl.BlockSpec((dh, do), lambda r: (0, 0)),
            pl.BlockSpec((1, do), lambda r: (0, 0)),
        ],
        out_specs=pl.BlockSpec((RB, do), lambda r: (r, 0)),
        out_shape=jax.ShapeDtypeStruct((NN, do), jnp.float32),
    )(z, colsum, m2, gamma, beta, w1, b1, w2, b2)


def kernel(x, edge_index, params):
    f32 = jnp.float32
    src = edge_index[0]
    dst = edge_index[1]
    NC, NS = _sc_info()
    NW = NC * NS
    nch = EE // NW // K
    src3 = src.reshape(NW, nch, K)
    dst3 = dst.reshape(NW, nch, K)
    rps = NPAD // NS

    ones128 = jnp.ones((K, 128), f32)
    zeros128 = jnp.zeros((rps, 128), f32)
    degs = _deg_sc(dst, ones128, zeros128)
    dinv = _tc_dinv(degs.reshape(NC, NPAD, 128))

    z = None
    colsum = None
    m2 = None
    htil = None
    for i in range(10):
        d = LAYER_DIMS[i + 1]
        dc = 128
        nc = (d + dc - 1) // dc
        if i == 0:
            htil = _tc_first(x, params['W0'], dinv, dc)
        else:
            htil = _tc_mid(
                z, colsum, m2,
                params['g%d' % (i - 1)].reshape(1, -1),
                params['be%d' % (i - 1)].reshape(1, -1),
                params['W%d' % i], dinv, nc, dc)
        tables = [htil[c] for c in range(nc)]
        S = _seg_sum_sc(tables, src, dst, zeros128, nc, dc)
        if nc == 1:
            S = [S] if not isinstance(S, (list, tuple)) else list(S)
        S = [s.reshape(NC, NPAD, dc) for s in S]
        z, colsum = _tc_combine(
            S, htil, dinv, params['bW%d' % i].reshape(1, -1), nc, dc, d)
        m2 = _tc_m2(z, colsum)

    return _tc_head(
        z, colsum, m2,
        params['g9'].reshape(1, -1), params['be9'].reshape(1, -1),
        params['lin1_W'], params['lin1_b'].reshape(1, -1),
        params['lin2_W'], params['lin2_b'].reshape(1, -1))
